# baseline (speedup 1.0000x reference)
import jax
import jax.numpy as jnp
import numpy as np
from jax.experimental import pallas as pl
from jax.experimental.pallas import tpu as pltpu

_B, _D, _L, _E = 64, 64, 1024, 1024
_N = _B * _D * _L

_SRANK = np.array([0, 7, 3, 5, 1, 6, 2, 4], dtype=np.int64)
_EIDX = np.arange(_E, dtype=np.int64)
_PERM = _SRANK[_EIDX & 7] * (_E // 8) + (_EIDX >> 3)


def _vq_body(z_ref, w_ref, q_ref, loss_ref):
    z = z_ref[0]
    w = w_ref[...]
    s2 = jax.lax.dot_general(
        -2.0 * w, z, (((1,), (0,)), ((), ())), preferred_element_type=jnp.float32
    )
    zsq = jnp.sum(z * z, axis=0, keepdims=True)
    wsq = jnp.sum(w * w, axis=1, keepdims=True)
    d = (zsq + wsq) + s2
    eiota = jax.lax.broadcasted_iota(jnp.int32, (_E, _L), 0)
    idx = jnp.argmin(d, axis=0).reshape(1, _L)
    onehot = (eiota == idx).astype(jnp.float32)
    q = jax.lax.dot_general(
        w, onehot, (((0,), (0,)), ((), ())),
        preferred_element_type=jnp.float32,
    )
    q_ref[0] = q
    diff = q - z
    loss_ref[0] = jnp.sum(diff * diff).reshape(1, 1)


@jax.jit
def kernel(z, W):
    q, loss_sum = pl.pallas_call(
        _vq_body,
        grid=(_B,),
        in_specs=[
            pl.BlockSpec((1, _D, _L), lambda b: (b, 0, 0)),
            pl.BlockSpec((_E, _D), lambda b: (0, 0)),
        ],
        out_specs=[
            pl.BlockSpec((1, _D, _L), lambda b: (b, 0, 0)),
            pl.BlockSpec((1, 1, 1), lambda b: (b, 0, 0)),
        ],
        out_shape=[
            jax.ShapeDtypeStruct((_B, _D, _L), jnp.float32),
            jax.ShapeDtypeStruct((_B, 1, 1), jnp.float32),
        ],
        compiler_params=pltpu.CompilerParams(
            dimension_semantics=("parallel",),
        ),
    )(z, W[_PERM])
    vq_loss = jnp.sum(loss_sum) / _N
    return q, vq_loss, 0.25 * vq_loss


# speedup vs baseline: 1.1030x; 1.1030x over previous
import jax
import jax.numpy as jnp
import numpy as np
from jax.experimental import pallas as pl
from jax.experimental.pallas import tpu as pltpu

_B, _D, _L, _E = 64, 64, 1024, 1024
_N = _B * _D * _L

_SRANK = np.array([0, 7, 3, 5, 1, 6, 2, 4], dtype=np.int64)
_EIDX = np.arange(_E, dtype=np.int64)
_PERM = _SRANK[_EIDX & 7] * (_E // 8) + (_EIDX >> 3)


_BB = 2


def _vq_body(z_ref, w_ref, q_ref, loss_ref):
    w = w_ref[...]
    w2 = -2.0 * w
    wsq = jnp.sum(w * w, axis=1, keepdims=True)
    eiota = jax.lax.broadcasted_iota(jnp.int32, (_E, _L), 0)
    for i in range(_BB):
        z = z_ref[i]
        s2 = jax.lax.dot_general(
            w2, z, (((1,), (0,)), ((), ())), preferred_element_type=jnp.float32
        )
        zsq = jnp.sum(z * z, axis=0, keepdims=True)
        d = (zsq + wsq) + s2
        idx = jnp.argmin(d, axis=0).reshape(1, _L)
        onehot = (eiota == idx).astype(jnp.float32)
        q = jax.lax.dot_general(
            w, onehot, (((0,), (0,)), ((), ())),
            preferred_element_type=jnp.float32,
        )
        q_ref[i] = q
        diff = q - z
        loss_ref[i] = jnp.sum(diff * diff).reshape(1, 1)


@jax.jit
def kernel(z, W):
    q, loss_sum = pl.pallas_call(
        _vq_body,
        grid=(_B // _BB,),
        in_specs=[
            pl.BlockSpec((_BB, _D, _L), lambda b: (b, 0, 0)),
            pl.BlockSpec((_E, _D), lambda b: (0, 0)),
        ],
        out_specs=[
            pl.BlockSpec((_BB, _D, _L), lambda b: (b, 0, 0)),
            pl.BlockSpec((_BB, 1, 1), lambda b: (b, 0, 0)),
        ],
        out_shape=[
            jax.ShapeDtypeStruct((_B, _D, _L), jnp.float32),
            jax.ShapeDtypeStruct((_B, 1, 1), jnp.float32),
        ],
        compiler_params=pltpu.CompilerParams(
            dimension_semantics=("parallel",),
        ),
    )(z, W[_PERM])
    vq_loss = jnp.sum(loss_sum) / _N
    return q, vq_loss, 0.25 * vq_loss


# speedup vs baseline: 1.1737x; 1.0641x over previous
import jax
import jax.numpy as jnp
import numpy as np
from jax.experimental import pallas as pl
from jax.experimental.pallas import tpu as pltpu

_B, _D, _L, _E = 64, 64, 1024, 1024
_N = _B * _D * _L

_SRANK = np.array([0, 7, 3, 5, 1, 6, 2, 4], dtype=np.int64)
_EIDX = np.arange(_E, dtype=np.int64)
_PERM = _SRANK[_EIDX & 7] * (_E // 8) + (_EIDX >> 3)


_BB = 4


def _vq_body(z_ref, w_ref, q_ref, loss_ref):
    w = w_ref[...]
    w2 = -2.0 * w
    wsq = jnp.sum(w * w, axis=1, keepdims=True)
    eiota = jax.lax.broadcasted_iota(jnp.int32, (_E, _L), 0)
    for i in range(_BB):
        z = z_ref[i]
        s2 = jax.lax.dot_general(
            w2, z, (((1,), (0,)), ((), ())), preferred_element_type=jnp.float32
        )
        zsq = jnp.sum(z * z, axis=0, keepdims=True)
        d = (zsq + wsq) + s2
        idx = jnp.argmin(d, axis=0).reshape(1, _L)
        onehot = (eiota == idx).astype(jnp.float32)
        q = jax.lax.dot_general(
            w, onehot, (((0,), (0,)), ((), ())),
            preferred_element_type=jnp.float32,
        )
        q_ref[i] = q
        diff = q - z
        loss_ref[i] = jnp.sum(diff * diff).reshape(1, 1)


@jax.jit
def kernel(z, W):
    q, loss_sum = pl.pallas_call(
        _vq_body,
        grid=(_B // _BB,),
        in_specs=[
            pl.BlockSpec((_BB, _D, _L), lambda b: (b, 0, 0)),
            pl.BlockSpec((_E, _D), lambda b: (0, 0)),
        ],
        out_specs=[
            pl.BlockSpec((_BB, _D, _L), lambda b: (b, 0, 0)),
            pl.BlockSpec((_BB, 1, 1), lambda b: (b, 0, 0)),
        ],
        out_shape=[
            jax.ShapeDtypeStruct((_B, _D, _L), jnp.float32),
            jax.ShapeDtypeStruct((_B, 1, 1), jnp.float32),
        ],
        compiler_params=pltpu.CompilerParams(
            dimension_semantics=("parallel",),
        ),
    )(z, W[_PERM])
    vq_loss = jnp.sum(loss_sum) / _N
    return q, vq_loss, 0.25 * vq_loss


# speedup vs baseline: 1.1991x; 1.0216x over previous
import jax
import jax.numpy as jnp
import numpy as np
from jax.experimental import pallas as pl
from jax.experimental.pallas import tpu as pltpu

_B, _D, _L, _E = 64, 64, 1024, 1024
_N = _B * _D * _L

_SRANK = np.array([0, 7, 3, 5, 1, 6, 2, 4], dtype=np.int64)
_EIDX = np.arange(_E, dtype=np.int64)
_PERM = _SRANK[_EIDX & 7] * (_E // 8) + (_EIDX >> 3)


_BB = 8


def _vq_body(z_ref, w_ref, q_ref, loss_ref):
    w = w_ref[...]
    w2 = -2.0 * w
    wsq = jnp.sum(w * w, axis=1, keepdims=True)
    eiota = jax.lax.broadcasted_iota(jnp.int32, (_E, _L), 0)
    for i in range(_BB):
        z = z_ref[i]
        s2 = jax.lax.dot_general(
            w2, z, (((1,), (0,)), ((), ())), preferred_element_type=jnp.float32
        )
        zsq = jnp.sum(z * z, axis=0, keepdims=True)
        d = (zsq + wsq) + s2
        idx = jnp.argmin(d, axis=0).reshape(1, _L)
        onehot = (eiota == idx).astype(jnp.float32)
        q = jax.lax.dot_general(
            w, onehot, (((0,), (0,)), ((), ())),
            preferred_element_type=jnp.float32,
        )
        q_ref[i] = q
        diff = q - z
        loss_ref[i] = jnp.sum(diff * diff).reshape(1, 1)


@jax.jit
def kernel(z, W):
    q, loss_sum = pl.pallas_call(
        _vq_body,
        grid=(_B // _BB,),
        in_specs=[
            pl.BlockSpec((_BB, _D, _L), lambda b: (b, 0, 0)),
            pl.BlockSpec((_E, _D), lambda b: (0, 0)),
        ],
        out_specs=[
            pl.BlockSpec((_BB, _D, _L), lambda b: (b, 0, 0)),
            pl.BlockSpec((_BB, 1, 1), lambda b: (b, 0, 0)),
        ],
        out_shape=[
            jax.ShapeDtypeStruct((_B, _D, _L), jnp.float32),
            jax.ShapeDtypeStruct((_B, 1, 1), jnp.float32),
        ],
        compiler_params=pltpu.CompilerParams(
            dimension_semantics=("parallel",),
        ),
    )(z, W[_PERM])
    vq_loss = jnp.sum(loss_sum) / _N
    return q, vq_loss, 0.25 * vq_loss


# speedup vs baseline: 1.2074x; 1.0069x over previous
import jax
import jax.numpy as jnp
import numpy as np
from jax.experimental import pallas as pl
from jax.experimental.pallas import tpu as pltpu

_B, _D, _L, _E = 64, 64, 1024, 1024
_N = _B * _D * _L

_SRANK = np.array([0, 7, 3, 5, 1, 6, 2, 4], dtype=np.int64)
_EIDX = np.arange(_E, dtype=np.int64)
_PERM = _SRANK[_EIDX & 7] * (_E // 8) + (_EIDX >> 3)


_BB = 16


def _vq_body(z_ref, w_ref, q_ref, loss_ref):
    w = w_ref[...]
    w2 = -2.0 * w
    wsq = jnp.sum(w * w, axis=1, keepdims=True)
    eiota = jax.lax.broadcasted_iota(jnp.int32, (_E, _L), 0)
    for i in range(_BB):
        z = z_ref[i]
        s2 = jax.lax.dot_general(
            w2, z, (((1,), (0,)), ((), ())), preferred_element_type=jnp.float32
        )
        zsq = jnp.sum(z * z, axis=0, keepdims=True)
        d = (zsq + wsq) + s2
        idx = jnp.argmin(d, axis=0).reshape(1, _L)
        onehot = (eiota == idx).astype(jnp.float32)
        q = jax.lax.dot_general(
            w, onehot, (((0,), (0,)), ((), ())),
            preferred_element_type=jnp.float32,
        )
        q_ref[i] = q
        diff = q - z
        loss_ref[i] = jnp.sum(diff * diff).reshape(1, 1)


@jax.jit
def kernel(z, W):
    q, loss_sum = pl.pallas_call(
        _vq_body,
        grid=(_B // _BB,),
        in_specs=[
            pl.BlockSpec((_BB, _D, _L), lambda b: (b, 0, 0)),
            pl.BlockSpec((_E, _D), lambda b: (0, 0)),
        ],
        out_specs=[
            pl.BlockSpec((_BB, _D, _L), lambda b: (b, 0, 0)),
            pl.BlockSpec((_BB, 1, 1), lambda b: (b, 0, 0)),
        ],
        out_shape=[
            jax.ShapeDtypeStruct((_B, _D, _L), jnp.float32),
            jax.ShapeDtypeStruct((_B, 1, 1), jnp.float32),
        ],
        compiler_params=pltpu.CompilerParams(
            dimension_semantics=("parallel",),
        ),
    )(z, W[_PERM])
    vq_loss = jnp.sum(loss_sum) / _N
    return q, vq_loss, 0.25 * vq_loss
